# baseline (speedup 1.0000x reference)
import functools
import math

import jax
import jax.numpy as jnp
from jax import lax
from jax.experimental import pallas as pl
from jax.experimental.pallas import tpu as pltpu
from jax.experimental.pallas import tpu_sc as plsc

_D = 64
_SCALE = math.sqrt(_D)
_NC = 2
_NS = 16
_NW = _NC * _NS
_LANES = 16

_IDX_W = 128
_CHUNK_ROWS = 1024
_K = _CHUNK_ROWS // _IDX_W


@functools.lru_cache(maxsize=None)
def _embed_kernel(B, b_per_w, n_chunks):
    mesh = plsc.VectorSubcoreMesh(core_axis_name="c", subcore_axis_name="s")

    @functools.partial(
        pl.kernel,
        mesh=mesh,
        out_type=jax.ShapeDtypeStruct((B, _D), jnp.float32),
        scratch_types=[
            pltpu.VMEM((_K, _IDX_W), jnp.int32),
            pltpu.VMEM((_CHUNK_ROWS, _D), jnp.float32),
            pltpu.SemaphoreType.DMA,
        ],
        compiler_params=pltpu.CompilerParams(use_tc_tiling_on_sc=False),
    )
    def k(idx_hbm, table_hbm, out_hbm, idx_v, rows_v, sem):
        wid = lax.axis_index("s") * _NC + lax.axis_index("c")
        base = wid * b_per_w

        def chunk_body(g, carry):
            r0 = pl.multiple_of(base + g * _CHUNK_ROWS, _CHUNK_ROWS)
            i0 = pl.multiple_of(r0 // _IDX_W, _K)
            pltpu.sync_copy(idx_hbm.at[pl.ds(i0, _K)], idx_v)
            copies = [
                pltpu.async_copy(
                    table_hbm.at[idx_v.at[j]],
                    rows_v.at[pl.ds(j * _IDX_W, _IDX_W)],
                    sem,
                )
                for j in range(_K)
            ]
            for cp in copies:
                cp.wait()

            def scale_row(i, c):
                for j in range(_D // _LANES):
                    sl = pl.ds(j * _LANES, _LANES)
                    rows_v[i, sl] = rows_v[i, sl] * _SCALE
                return c

            lax.fori_loop(0, _CHUNK_ROWS, scale_row, 0)
            pltpu.sync_copy(rows_v, out_hbm.at[pl.ds(r0, _CHUNK_ROWS)])
            return carry

        lax.fori_loop(0, n_chunks, chunk_body, 0)

    return k


def kernel(inputs, table):
    S, T = inputs.shape
    B = S * T
    b_per_w = B // _NW
    n_chunks = b_per_w // _CHUNK_ROWS
    idx2d = inputs.reshape(B // _IDX_W, _IDX_W)
    out = _embed_kernel(B, b_per_w, n_chunks)(idx2d, table)
    return out.reshape(S, T, _D)


# speedup vs baseline: 1.1111x; 1.1111x over previous
import functools
import math

import jax
import jax.numpy as jnp
from jax import lax
from jax.experimental import pallas as pl
from jax.experimental.pallas import tpu as pltpu
from jax.experimental.pallas import tpu_sc as plsc

_D = 64
_SCALE = math.sqrt(_D)
_NC = 2
_NS = 16
_NW = _NC * _NS
_LANES = 16

_IDX_W = 128
_CHUNK = 256
_K = _CHUNK // _IDX_W
_NBUF = 4


@functools.lru_cache(maxsize=None)
def _embed_kernel(B):
    b_per_w = B // _NW
    n_idx_rows = b_per_w // _IDX_W
    n_chunks = b_per_w // _CHUNK
    n_super = n_chunks // _NBUF
    assert b_per_w % _CHUNK == 0 and n_chunks % _NBUF == 0

    mesh = plsc.VectorSubcoreMesh(core_axis_name="c", subcore_axis_name="s")

    @functools.partial(
        pl.kernel,
        mesh=mesh,
        out_type=jax.ShapeDtypeStruct((B, _D), jnp.float32),
        scratch_types=[
            pltpu.VMEM((n_idx_rows, _IDX_W), jnp.int32),
            pltpu.VMEM((_NBUF * _CHUNK, _D), jnp.float32),
        ]
        + [pltpu.SemaphoreType.DMA] * (2 * _NBUF),
        compiler_params=pltpu.CompilerParams(use_tc_tiling_on_sc=False),
    )
    def k(idx_hbm, table_hbm, out_hbm, idx_all, rows_v, *sems):
        sg = sems[:_NBUF]
        ss = sems[_NBUF:]
        wid = lax.axis_index("s") * _NC + lax.axis_index("c")
        base = pl.multiple_of(wid * b_per_w, b_per_w)
        ib = pl.multiple_of(wid * n_idx_rows, 8)
        pltpu.sync_copy(idx_hbm.at[pl.ds(ib, n_idx_rows)], idx_all)

        def gather_copy(g, b):
            return [
                pltpu.make_async_copy(
                    table_hbm.at[idx_all.at[g * _K + j]],
                    rows_v.at[pl.ds(b * _CHUNK + j * _IDX_W, _IDX_W)],
                    sg[b],
                )
                for j in range(_K)
            ]

        def store_copy(g, b):
            r0 = pl.multiple_of(base + g * _CHUNK, _CHUNK)
            return pltpu.make_async_copy(
                rows_v.at[pl.ds(b * _CHUNK, _CHUNK)],
                out_hbm.at[pl.ds(r0, _CHUNK)],
                ss[b],
            )

        for cp in gather_copy(0, 0):
            cp.start()
        for cp in gather_copy(1, 1):
            cp.start()

        def super_body(s, carry):
            for b in range(_NBUF):
                g = _NBUF * s + b
                bw = (b + 2) % _NBUF

                if b >= 2:
                    store_copy(g - 2, bw).wait()
                else:
                    @pl.when(s >= 1)
                    def _():
                        store_copy(g - 2, bw).wait()

                if b < 2:
                    for cp in gather_copy(g + 2, bw):
                        cp.start()
                else:
                    @pl.when(s < n_super - 1)
                    def _():
                        for cp in gather_copy(g + 2, bw):
                            cp.start()

                for cp in gather_copy(g, b):
                    cp.wait()

                @plsc.parallel_loop(0, _CHUNK, unroll=8)
                def _(i):
                    for j in range(_D // _LANES):
                        sl = pl.ds(j * _LANES, _LANES)
                        rows_v[b * _CHUNK + i, sl] = (
                            rows_v[b * _CHUNK + i, sl] * _SCALE
                        )

                store_copy(g, b).start()
            return carry

        lax.fori_loop(0, n_super, super_body, 0)
        store_copy(n_chunks - 2, (n_chunks - 2) % _NBUF).wait()
        store_copy(n_chunks - 1, (n_chunks - 1) % _NBUF).wait()

    return k


def kernel(inputs, table):
    S, T = inputs.shape
    B = S * T
    idx2d = inputs.reshape(B // _IDX_W, _IDX_W)
    out = _embed_kernel(B)(idx2d, table)
    return out.reshape(S, T, _D)
